# baseline (speedup 1.0000x reference)
import functools

import jax
import jax.numpy as jnp
from jax import lax
from jax.experimental import pallas as pl
from jax.experimental.pallas import tpu as pltpu
from jax.experimental.pallas import tpu_sc as plsc

_NUM_STATES = 8192
_N_SEQ = 4096
_SEQ_LEN = 512
_TOTAL = _N_SEQ * _SEQ_LEN
_NW = 32
_PER_W = _TOTAL // _NW
_CHUNK = 8192
_NCHUNK = _PER_W // _CHUNK
_GROUPS = _CHUNK // 128


def _sc_body(seq_hbm, table_hbm, init_hbm, out_hbm, seq_v, idx_v, val_v, ini_v, sem):
    wid = lax.axis_index("s") * 2 + lax.axis_index("c")
    seq_v[pl.ds(_CHUNK, 16)] = jnp.zeros((16,), jnp.int32)
    lane = lax.iota(jnp.int32, 16)

    def chunk_body(c, carry):
        base = wid * _PER_W + c * _CHUNK
        pltpu.sync_copy(seq_hbm.at[pl.ds(base, _CHUNK)], seq_v.at[pl.ds(0, _CHUNK)])

        def idx_body(i, carry2):
            o = i * 16
            a = seq_v[pl.ds(o, 16)]
            b = seq_v[pl.ds(o + 1, 16)]
            idx_v[pl.ds(o, 16)] = a * _NUM_STATES + b
            return carry2

        lax.fori_loop(0, _CHUNK // 16, idx_body, 0)

        s0 = plsc.load_gather(seq_v, [lane * _SEQ_LEN])

        def fire(j, carry2):
            pltpu.make_async_copy(
                table_hbm.at[idx_v.at[pl.ds(j * 128, 128)]], val_v.at[j], sem
            ).start()
            return carry2

        lax.fori_loop(0, _GROUPS, fire, 0)
        pltpu.make_async_copy(init_hbm.at[s0], ini_v, sem).start()

        def drain(j, carry2):
            pltpu.make_async_copy(
                table_hbm.at[idx_v.at[pl.ds(j * 128, 128)]], val_v.at[j], sem
            ).wait()
            return carry2

        lax.fori_loop(0, _GROUPS, drain, 0)
        pltpu.make_async_copy(init_hbm.at[s0], ini_v, sem).wait()

        plsc.store_scatter(
            val_v, [lane * 4 + 3, jnp.full((16,), 127, jnp.int32)], ini_v[...]
        )
        pltpu.sync_copy(
            val_v, out_hbm.at[pl.ds((wid * _NCHUNK + c) * _GROUPS, _GROUPS)]
        )
        return carry

    lax.fori_loop(0, _NCHUNK, chunk_body, 0)


_sc_gather = functools.partial(
    pl.kernel,
    mesh=plsc.VectorSubcoreMesh(core_axis_name="c", subcore_axis_name="s"),
    compiler_params=pltpu.CompilerParams(needs_layout_passes=False),
    out_type=jax.ShapeDtypeStruct((_TOTAL // 128, 128), jnp.float32),
    scratch_types=[
        pltpu.VMEM((_CHUNK + 16,), jnp.int32),
        pltpu.VMEM((_CHUNK,), jnp.int32),
        pltpu.VMEM((_GROUPS, 128), jnp.float32),
        pltpu.VMEM((16,), jnp.float32),
        pltpu.SemaphoreType.DMA,
    ],
)(_sc_body)


def _logsum_body(x_ref, o_ref):
    i = pl.program_id(0)

    @pl.when(i == 0)
    def _():
        o_ref[0, 0] = 0.0

    o_ref[0, 0] += jnp.sum(jnp.log(x_ref[...]))


def _logsum(x):
    return pl.pallas_call(
        _logsum_body,
        grid=(32,),
        in_specs=[pl.BlockSpec((512, 128), lambda i: (i, 0))],
        out_specs=pl.BlockSpec(memory_space=pltpu.SMEM),
        out_shape=jax.ShapeDtypeStruct((1, 1), jnp.float32),
    )(x)


def kernel(sequences, initial_probs, transition_probs):
    seq_flat = sequences.reshape(-1)
    table_flat = transition_probs.reshape(-1)
    gathered = _sc_gather(seq_flat, table_flat, initial_probs)
    total = _logsum(gathered)
    return total[0, 0] / jnp.float32(_N_SEQ)


# speedup vs baseline: 1.1263x; 1.1263x over previous
import functools

import jax
import jax.numpy as jnp
from jax import lax
from jax.experimental import pallas as pl
from jax.experimental.pallas import tpu as pltpu
from jax.experimental.pallas import tpu_sc as plsc

_NUM_STATES = 8192
_N_SEQ = 4096
_SEQ_LEN = 512
_TOTAL = _N_SEQ * _SEQ_LEN
_NW = 32
_PER_W = _TOTAL // _NW
_CHUNK = 8192
_NCHUNK = _PER_W // _CHUNK
_GROUPS = _CHUNK // 128
_ITERS = _CHUNK // 16

_LN2 = 0.6931471805599453
_MANT = 0x007FFFFF
_ONE_BITS = 0x3F800000


def _ln_residual(m):
    t = (m - 1.0) / (m + 1.0)
    t2 = t * t
    p = 1.0 / 9.0
    p = p * t2 + 1.0 / 7.0
    p = p * t2 + 1.0 / 5.0
    p = p * t2 + 1.0 / 3.0
    p = p * t2 + 1.0
    return 2.0 * t * p


def _sc_body(
    seq_hbm, table_hbm, init_hbm, out_hbm,
    seq_v, idx_v0, idx_v1, val_v0, val_v1, ini_v0, ini_v1, acc_v,
    sem0, sem1,
):
    wid = lax.axis_index("s") * 2 + lax.axis_index("c")
    seq_v[pl.ds(_CHUNK, 16)] = jnp.zeros((16,), jnp.int32)
    lane = lax.iota(jnp.int32, 16)

    idx_bufs = (idx_v0, idx_v1)
    val_bufs = (val_v0, val_v1)
    ini_bufs = (ini_v0, ini_v1)
    sems = (sem0, sem1)

    def stage(c):
        idx_v, val_v, ini_v, sem = (
            idx_bufs[c % 2], val_bufs[c % 2], ini_bufs[c % 2], sems[c % 2]
        )
        base = wid * _PER_W + c * _CHUNK
        pltpu.sync_copy(seq_hbm.at[pl.ds(base, _CHUNK)], seq_v.at[pl.ds(0, _CHUNK)])

        def idx_body(i, carry):
            o = i * 16
            a = seq_v[pl.ds(o, 16)]
            b = seq_v[pl.ds(o + 1, 16)]
            idx_v[pl.ds(o, 16)] = a * _NUM_STATES + b
            return carry

        lax.fori_loop(0, _ITERS, idx_body, 0)

        def fire(j, carry):
            pltpu.make_async_copy(
                table_hbm.at[idx_v.at[pl.ds(j * 128, 128)]],
                val_v.at[pl.ds(j * 128, 128)],
                sem,
            ).start()
            return carry

        lax.fori_loop(0, _GROUPS, fire, 0)
        s0 = plsc.load_gather(seq_v, [lane * _SEQ_LEN])
        pltpu.make_async_copy(init_hbm.at[s0], ini_v, sem).start()

    def consume(c, state):
        idx_v, val_v, ini_v, sem = (
            idx_bufs[c % 2], val_bufs[c % 2], ini_bufs[c % 2], sems[c % 2]
        )

        def drain(j, carry):
            pltpu.make_async_copy(
                table_hbm.at[idx_v.at[pl.ds(j * 128, 128)]],
                val_v.at[pl.ds(j * 128, 128)],
                sem,
            ).wait()
            return carry

        lax.fori_loop(0, _GROUPS, drain, 0)
        s0 = plsc.load_gather(seq_v, [lane * _SEQ_LEN])
        pltpu.make_async_copy(init_hbm.at[s0], ini_v, sem).wait()

        plsc.store_scatter(val_v, [lane * _SEQ_LEN + 511], ini_v[...])

        def prod_body(i, st):
            macc, eacc, vmin = st
            v = plsc.load_gather(val_v, [i * 16 + lane])
            m2 = macc * v
            bits = plsc.bitcast(m2, jnp.int32)
            eacc = eacc + (bits >> 23)
            macc = plsc.bitcast((bits & _MANT) | _ONE_BITS, jnp.float32)
            vmin = jnp.minimum(vmin, v)
            return macc, eacc, vmin

        return lax.fori_loop(0, _ITERS, prod_body, state)

    state = (
        jnp.ones((16,), jnp.float32),
        jnp.zeros((16,), jnp.int32),
        jnp.full((16,), jnp.inf, jnp.float32),
    )
    stage(0)
    for c in range(1, _NCHUNK):
        stage(c)
        state = consume(c - 1, state)
    state = consume(_NCHUNK - 1, state)

    macc, eacc, vmin = state
    n_per_lane = _ITERS * _NCHUNK
    ln_part = (eacc - 127 * n_per_lane).astype(jnp.float32) * _LN2 + _ln_residual(macc)
    acc_v[...] = jnp.where(vmin == 0.0, jnp.float32(-jnp.inf), ln_part)
    pltpu.sync_copy(acc_v, out_hbm.at[pl.ds(wid * 16, 16)])


_sc_sumlog = functools.partial(
    pl.kernel,
    mesh=plsc.VectorSubcoreMesh(core_axis_name="c", subcore_axis_name="s"),
    compiler_params=pltpu.CompilerParams(needs_layout_passes=False),
    out_type=jax.ShapeDtypeStruct((_NW * 16,), jnp.float32),
    scratch_types=[
        pltpu.VMEM((_CHUNK + 16,), jnp.int32),
        pltpu.VMEM((_CHUNK,), jnp.int32),
        pltpu.VMEM((_CHUNK,), jnp.int32),
        pltpu.VMEM((_CHUNK,), jnp.float32),
        pltpu.VMEM((_CHUNK,), jnp.float32),
        pltpu.VMEM((16,), jnp.float32),
        pltpu.VMEM((16,), jnp.float32),
        pltpu.VMEM((16,), jnp.float32),
        pltpu.SemaphoreType.DMA,
        pltpu.SemaphoreType.DMA,
    ],
)(_sc_body)


def kernel(sequences, initial_probs, transition_probs):
    seq_flat = sequences.reshape(-1)
    table_flat = transition_probs.reshape(-1)
    partials = _sc_sumlog(seq_flat, table_flat, initial_probs)
    return jnp.sum(partials) / jnp.float32(_N_SEQ)


# speedup vs baseline: 1.1638x; 1.0333x over previous
import functools

import jax
import jax.numpy as jnp
from jax import lax
from jax.experimental import pallas as pl
from jax.experimental.pallas import tpu as pltpu
from jax.experimental.pallas import tpu_sc as plsc

_NUM_STATES = 8192
_N_SEQ = 4096
_SEQ_LEN = 512
_TOTAL = _N_SEQ * _SEQ_LEN
_NW = 32
_PER_W = _TOTAL // _NW
_CHUNK = 8192
_NCHUNK = _PER_W // _CHUNK
_GROUPS = _CHUNK // 128
_ITERS = _CHUNK // 16

_LN2 = 0.6931471805599453
_MANT = 0x007FFFFF
_ONE_BITS = 0x3F800000


def _idx_build_body(seq_ref, out_ref):
    x = seq_ref[...]
    shifted = jnp.concatenate(
        [x[:, 1:], jnp.zeros((128, 1), jnp.int32)], axis=1
    )
    col = jax.lax.broadcasted_iota(jnp.int32, (128, _SEQ_LEN), 1)
    idx = jnp.where(col == _SEQ_LEN - 1, x[:, 0:1], x * _NUM_STATES + shifted)
    out_ref[...] = idx.reshape(512, 128)


def _idx_build(sequences):
    return pl.pallas_call(
        _idx_build_body,
        grid=(_N_SEQ // 128,),
        in_specs=[pl.BlockSpec((128, _SEQ_LEN), lambda i: (i, 0))],
        out_specs=pl.BlockSpec((512, 128), lambda i: (i, 0)),
        out_shape=jax.ShapeDtypeStruct((_TOTAL // 128, 128), jnp.int32),
    )(sequences)


def _ln_residual(m):
    t = (m - 1.0) / (m + 1.0)
    t2 = t * t
    p = 1.0 / 9.0
    p = p * t2 + 1.0 / 7.0
    p = p * t2 + 1.0 / 5.0
    p = p * t2 + 1.0 / 3.0
    p = p * t2 + 1.0
    return 2.0 * t * p


def _sc_body(
    idx_hbm, table_hbm, init_hbm, out_hbm,
    idx_v0, idx_v1, val_v0, val_v1, ini_v0, ini_v1, acc_v,
    sem0, sem1,
):
    wid = lax.axis_index("s") * 2 + lax.axis_index("c")
    lane = lax.iota(jnp.int32, 16)
    c127 = jnp.full((16,), 127, jnp.int32)

    idx_bufs = (idx_v0, idx_v1)
    val_bufs = (val_v0, val_v1)
    ini_bufs = (ini_v0, ini_v1)
    sems = (sem0, sem1)

    def stage(c):
        idx_v, val_v, ini_v, sem = (
            idx_bufs[c % 2], val_bufs[c % 2], ini_bufs[c % 2], sems[c % 2]
        )
        row0 = (wid * _NCHUNK + c) * _GROUPS
        pltpu.sync_copy(idx_hbm.at[pl.ds(row0, _GROUPS)], idx_v)

        def fire(j, carry):
            pltpu.make_async_copy(
                table_hbm.at[idx_v.at[j]],
                val_v.at[pl.ds(j * 128, 128)],
                sem,
            ).start()
            return carry

        lax.fori_loop(0, _GROUPS, fire, 0)
        s0 = plsc.load_gather(idx_v, [lane * 4 + 3, c127])
        pltpu.make_async_copy(init_hbm.at[s0], ini_v, sem).start()

    def consume(c, state):
        idx_v, val_v, ini_v, sem = (
            idx_bufs[c % 2], val_bufs[c % 2], ini_bufs[c % 2], sems[c % 2]
        )

        def drain(j, carry):
            pltpu.make_async_copy(
                table_hbm.at[idx_v.at[j]],
                val_v.at[pl.ds(j * 128, 128)],
                sem,
            ).wait()
            return carry

        lax.fori_loop(0, _GROUPS, drain, 0)
        s0 = plsc.load_gather(idx_v, [lane * 4 + 3, c127])
        pltpu.make_async_copy(init_hbm.at[s0], ini_v, sem).wait()

        plsc.store_scatter(val_v, [lane * _SEQ_LEN + 511], ini_v[...])

        def prod_body(i, st):
            macc, eacc, vmin = st
            v = plsc.load_gather(val_v, [i * 16 + lane])
            m2 = macc * v
            bits = plsc.bitcast(m2, jnp.int32)
            eacc = eacc + (bits >> 23)
            macc = plsc.bitcast((bits & _MANT) | _ONE_BITS, jnp.float32)
            vmin = jnp.minimum(vmin, v)
            return macc, eacc, vmin

        return lax.fori_loop(0, _ITERS, prod_body, state)

    state = (
        jnp.ones((16,), jnp.float32),
        jnp.zeros((16,), jnp.int32),
        jnp.full((16,), jnp.inf, jnp.float32),
    )
    stage(0)
    for c in range(1, _NCHUNK):
        stage(c)
        state = consume(c - 1, state)
    state = consume(_NCHUNK - 1, state)

    macc, eacc, vmin = state
    n_per_lane = _ITERS * _NCHUNK
    ln_part = (eacc - 127 * n_per_lane).astype(jnp.float32) * _LN2 + _ln_residual(macc)
    acc_v[...] = jnp.where(vmin == 0.0, jnp.float32(-jnp.inf), ln_part)
    pltpu.sync_copy(acc_v, out_hbm.at[pl.ds(wid * 16, 16)])


_sc_sumlog = functools.partial(
    pl.kernel,
    mesh=plsc.VectorSubcoreMesh(core_axis_name="c", subcore_axis_name="s"),
    compiler_params=pltpu.CompilerParams(needs_layout_passes=False),
    out_type=jax.ShapeDtypeStruct((_NW * 16,), jnp.float32),
    scratch_types=[
        pltpu.VMEM((_GROUPS, 128), jnp.int32),
        pltpu.VMEM((_GROUPS, 128), jnp.int32),
        pltpu.VMEM((_CHUNK,), jnp.float32),
        pltpu.VMEM((_CHUNK,), jnp.float32),
        pltpu.VMEM((16,), jnp.float32),
        pltpu.VMEM((16,), jnp.float32),
        pltpu.VMEM((16,), jnp.float32),
        pltpu.SemaphoreType.DMA,
        pltpu.SemaphoreType.DMA,
    ],
)(_sc_body)


def kernel(sequences, initial_probs, transition_probs):
    table_flat = transition_probs.reshape(-1)
    idxflat = _idx_build(sequences)
    partials = _sc_sumlog(idxflat, table_flat, initial_probs)
    return jnp.sum(partials) / jnp.float32(_N_SEQ)
